# baseline (speedup 1.0000x reference)
import jax
import jax.numpy as jnp
from jax import lax
from jax.experimental import pallas as pl
from jax.experimental.pallas import tpu as pltpu
from jax.experimental.pallas import tpu_sc as plsc

T = 2048
D = 2048
E = 8
W = 1024
K = 2
BLK = 128
S = 5120
NB = S // BLK

NC = 2
NS = 16
NW = NC * NS

_HD = D // 2


def _bf16_bits(xf32):
    t = lax.bitcast_convert_type(xf32, jnp.int32)
    carry = jnp.bitwise_and(lax.shift_right_logical(t, 16), 1)
    return lax.shift_right_logical(t + 32767 + carry, 16)


def _router_body(x_ref, wr_ref, p0_ref, p1_ref, cw0_ref, cw1_ref,
                 blke_ref, z_ref, xp_ref):
    x = x_ref[...]
    wr = wr_ref[...]
    lo = _bf16_bits(x[:, :_HD])
    hi = _bf16_bits(x[:, _HD:])
    xp_ref[...] = jnp.bitwise_or(lo, lax.shift_left(hi, 16))
    logits = lax.dot_general(x, wr, (((1,), (0,)), ((), ())),
                             preferred_element_type=jnp.float32)
    eidx = lax.broadcasted_iota(jnp.int32, (T, E), 1)
    m1 = jnp.max(logits, axis=1, keepdims=True)
    a1 = jnp.min(jnp.where(logits == m1, eidx, E), axis=1, keepdims=True)
    logits2 = jnp.where(eidx == a1, -jnp.inf, logits)
    m2 = jnp.max(logits2, axis=1, keepdims=True)
    a2 = jnp.min(jnp.where(logits2 == m2, eidx, E), axis=1, keepdims=True)
    s1 = jax.nn.sigmoid(m1)
    s2 = jax.nn.sigmoid(m2)
    denom = s1 + s2 + 1e-20
    cw0_ref[...] = s1 / denom
    cw1_ref[...] = s2 / denom
    oh1 = eidx == a1
    oh2 = eidx == a2
    ohsum = oh1.astype(jnp.float32) + oh2.astype(jnp.float32)
    cnt = jnp.sum(ohsum, axis=0, keepdims=True)
    pc = jnp.ceil(cnt / BLK) * BLK
    ei = lax.broadcasted_iota(jnp.int32, (E, E), 0)
    ej = lax.broadcasted_iota(jnp.int32, (E, E), 1)
    strict = (ei < ej).astype(jnp.float32)
    starts = lax.dot_general(pc, strict, (((1,), (0,)), ((), ())),
                             preferred_element_type=jnp.float32)
    ti = lax.broadcasted_iota(jnp.int32, (T, 1), 0)
    tj = lax.broadcasted_iota(jnp.int32, (1, T), 1)
    lmask = (tj < ti).astype(jnp.bfloat16)
    cex = lax.dot_general(lmask, ohsum.astype(jnp.bfloat16),
                          (((1,), (0,)), ((), ())),
                          preferred_element_type=jnp.float32)
    pos_base = cex + starts
    p0_ref[...] = jnp.sum(jnp.where(oh1, pos_base, 0.0), axis=1,
                          keepdims=True).astype(jnp.int32)
    p1_ref[...] = jnp.sum(jnp.where(oh2, pos_base, 0.0), axis=1,
                          keepdims=True).astype(jnp.int32)
    bi = lax.broadcasted_iota(jnp.int32, (NB, E), 0).astype(jnp.float32) * BLK
    blke_ref[...] = (jnp.sum((bi >= starts).astype(jnp.int32), axis=1,
                             keepdims=True) - 1)
    mx = jnp.max(logits, axis=1, keepdims=True)
    lse = mx + jnp.log(jnp.sum(jnp.exp(logits - mx), axis=1, keepdims=True))
    z_ref[...] = jnp.mean(lse * lse).reshape(1, 1)


def _router_call(x_flat, W_router):
    return pl.pallas_call(
        _router_body,
        out_shape=(
            jax.ShapeDtypeStruct((T, 1), jnp.int32),
            jax.ShapeDtypeStruct((T, 1), jnp.int32),
            jax.ShapeDtypeStruct((T, 1), jnp.float32),
            jax.ShapeDtypeStruct((T, 1), jnp.float32),
            jax.ShapeDtypeStruct((NB, 1), jnp.int32),
            jax.ShapeDtypeStruct((1, 1), jnp.float32),
            jax.ShapeDtypeStruct((T, _HD), jnp.int32),
        ),
    )(x_flat, W_router)


def _slots_body(p0_ref, p1_ref, cw0_ref, cw1_ref, tok_ref, cw_ref):
    j = pl.program_id(0)
    p0r = p0_ref[...]
    p1r = p1_ref[...]
    cw0r = cw0_ref[...]
    cw1r = cw1_ref[...]
    sid = lax.broadcasted_iota(jnp.int32, (BLK, T), 0) + j * BLK
    tok = lax.broadcasted_iota(jnp.int32, (BLK, T), 1)
    m0 = sid == p0r
    m1 = sid == p1r
    tok_ref[...] = (jnp.sum(jnp.where(m0, tok, 0), axis=1, keepdims=True)
                    + jnp.sum(jnp.where(m1, tok, 0), axis=1, keepdims=True))
    cw_ref[...] = (jnp.sum(jnp.where(m0, cw0r, 0.0), axis=1, keepdims=True)
                   + jnp.sum(jnp.where(m1, cw1r, 0.0), axis=1, keepdims=True))


def _slots_call(p0_row, p1_row, cw0_row, cw1_row):
    return pl.pallas_call(
        _slots_body,
        grid=(NB,),
        in_specs=[pl.BlockSpec((1, T), lambda j: (0, 0))] * 4,
        out_specs=(
            pl.BlockSpec((BLK, 1), lambda j: (j, 0)),
            pl.BlockSpec((BLK, 1), lambda j: (j, 0)),
        ),
        out_shape=(
            jax.ShapeDtypeStruct((S, 1), jnp.int32),
            jax.ShapeDtypeStruct((S, 1), jnp.float32),
        ),
    )(p0_row, p1_row, cw0_row, cw1_row)


_CROWS = 16
_RPW = S // NW
_NCH = _RPW // _CROWS
_NBUF = 4


def _gather_body(x_hbm, idx_hbm, out_hbm, idx_v, b0, b1, b2, b3,
                 sg0, sg1, sg2, sg3, sw0, sw1, sw2, sw3):
    wid = lax.axis_index("s") * NC + lax.axis_index("c")
    base = wid * _RPW
    pltpu.sync_copy(idx_hbm.at[pl.ds(base, _RPW)], idx_v)
    bufs = [b0, b1, b2, b3]
    gsems = [sg0, sg1, sg2, sg3]
    wsems = [sw0, sw1, sw2, sw3]

    def gath(c):
        return pltpu.async_copy(
            x_hbm.at[idx_v.at[pl.ds(c * _CROWS, _CROWS)]],
            bufs[c % _NBUF], gsems[c % _NBUF])

    def wr(c):
        return pltpu.async_copy(
            bufs[c % _NBUF], out_hbm.at[pl.ds(base + c * _CROWS, _CROWS)],
            wsems[c % _NBUF])

    gd = [None] * _NCH
    wd = [None] * _NCH
    for c in range(_NBUF):
        gd[c] = gath(c)
    for c in range(_NCH):
        gd[c].wait()
        wd[c] = wr(c)
        n = c + _NBUF
        if n < _NCH:
            wd[c].wait()
            gd[n] = gath(n)
    for c in range(_NCH - _NBUF, _NCH):
        wd[c].wait()


def _gather_rows(x_packed, slot_tok):
    mesh = plsc.VectorSubcoreMesh(core_axis_name="c", subcore_axis_name="s",
                                  num_cores=NC, num_subcores=NS)
    fn = pl.kernel(
        _gather_body,
        out_type=jax.ShapeDtypeStruct((S, _HD), jnp.int32),
        mesh=mesh,
        scratch_types=(
            [pltpu.VMEM((_RPW,), jnp.int32)]
            + [pltpu.VMEM((_CROWS, _HD), jnp.int32)] * _NBUF
            + [pltpu.SemaphoreType.DMA] * (2 * _NBUF)
        ),
    )
    return fn(x_packed, slot_tok)


def _mlp_body(blke_ref, xs_ref, w1_ref, w2_ref, cw_ref, out_ref):
    xp = xs_ref[...]
    xlo = lax.bitcast_convert_type(lax.shift_left(xp, 16), jnp.float32)
    xhi = lax.bitcast_convert_type(
        jnp.bitwise_and(xp, jnp.int32(-65536)), jnp.float32)
    xs = jnp.concatenate([xlo, xhi], axis=1)
    h = lax.dot_general(xs, w1_ref[...], (((1,), (0,)), ((), ())),
                        preferred_element_type=jnp.float32)
    h = jnp.square(jnp.maximum(h, 0.0))
    o = lax.dot_general(h, w2_ref[...], (((1,), (0,)), ((), ())),
                        preferred_element_type=jnp.float32)
    out_ref[...] = o * cw_ref[...]


def _mlp_call(blke, xs_packed, w1, w2, slot_cw):
    grid_spec = pltpu.PrefetchScalarGridSpec(
        num_scalar_prefetch=1,
        grid=(NB,),
        in_specs=[
            pl.BlockSpec((BLK, _HD), lambda b, blke: (b, 0)),
            pl.BlockSpec((D, W), lambda b, blke: (0, blke[b])),
            pl.BlockSpec((W, D), lambda b, blke: (blke[b], 0)),
            pl.BlockSpec((BLK, 1), lambda b, blke: (b, 0)),
        ],
        out_specs=pl.BlockSpec((BLK, D), lambda b, blke: (b, 0)),
    )
    return pl.pallas_call(
        _mlp_body,
        grid_spec=grid_spec,
        out_shape=jax.ShapeDtypeStruct((S, D), jnp.float32),
    )(blke, xs_packed, w1, w2, slot_cw)


_CTOK = 8
_TPW = T // NW
_NCHC = _TPW // _CTOK
_NBUFC = 2


def _combine_body(part_hbm, p0_hbm, p1_hbm, out_hbm, p0_v, p1_v,
                  a0, a1, g0, g1, sa0, sa1, sg0, sg1, sw0, sw1):
    wid = lax.axis_index("s") * NC + lax.axis_index("c")
    base = wid * _TPW
    pltpu.sync_copy(p0_hbm.at[pl.ds(base, _TPW)], p0_v)
    pltpu.sync_copy(p1_hbm.at[pl.ds(base, _TPW)], p1_v)
    accs, gs = [a0, a1], [g0, g1]
    asems, gsems, wsems = [sa0, sa1], [sg0, sg1], [sw0, sw1]

    def gath(c):
        k = c % _NBUFC
        sl = pl.ds(c * _CTOK, _CTOK)
        return (pltpu.async_copy(part_hbm.at[p0_v.at[sl]], accs[k], asems[k]),
                pltpu.async_copy(part_hbm.at[p1_v.at[sl]], gs[k], gsems[k]))

    def wr(c):
        k = c % _NBUFC
        return pltpu.async_copy(
            accs[k], out_hbm.at[pl.ds(base + c * _CTOK, _CTOK)], wsems[k])

    gd = [None] * _NCHC
    wd = [None] * _NCHC
    for c in range(_NBUFC):
        gd[c] = gath(c)
    for c in range(_NCHC):
        k = c % _NBUFC
        gd[c][0].wait()
        gd[c][1].wait()
        acc_v, g_v = accs[k], gs[k]
        for r in range(_CTOK):
            def add16(ci, carry2, r=r, acc_v=acc_v, g_v=g_v):
                sl = pl.ds(ci * 16, 16)
                acc_v[r, sl] = acc_v[r, sl] + g_v[r, sl]
                return carry2
            lax.fori_loop(0, D // 16, add16, 0, unroll=8)
        wd[c] = wr(c)
        n = c + _NBUFC
        if n < _NCHC:
            wd[c].wait()
            gd[n] = gath(n)
    for c in range(_NCHC - _NBUFC, _NCHC):
        wd[c].wait()


def _combine_rows(partial, p0, p1):
    mesh = plsc.VectorSubcoreMesh(core_axis_name="c", subcore_axis_name="s",
                                  num_cores=NC, num_subcores=NS)
    fn = pl.kernel(
        _combine_body,
        out_type=jax.ShapeDtypeStruct((T, D), jnp.float32),
        mesh=mesh,
        scratch_types=(
            [pltpu.VMEM((_TPW,), jnp.int32)] * 2
            + [pltpu.VMEM((_CTOK, D), jnp.float32)] * (2 * _NBUFC)
            + [pltpu.SemaphoreType.DMA] * (3 * _NBUFC)
        ),
    )
    return fn(partial, p0, p1)


def kernel(x, W_router, w1, w2):
    b, s, d = x.shape
    x_flat = x.reshape(T, D)
    pos0, pos1, cw0, cw1, blke, z, x_packed = _router_call(x_flat, W_router)
    slot_tok_col, slot_cw = _slots_call(pos0.reshape(1, T), pos1.reshape(1, T),
                                        cw0.reshape(1, T), cw1.reshape(1, T))
    slot_tok = slot_tok_col.reshape(S)
    xs_packed = _gather_rows(x_packed, slot_tok)
    partial = _mlp_call(blke.reshape(NB), xs_packed, w1, w2, slot_cw)
    out_flat = _combine_rows(partial, pos0.reshape(T), pos1.reshape(T))
    return out_flat.reshape(b, s, d), z.reshape(())


# speedup vs baseline: 1.0117x; 1.0117x over previous
import functools

import jax
import jax.numpy as jnp
from jax import lax
from jax.experimental import pallas as pl
from jax.experimental.pallas import tpu as pltpu
from jax.experimental.pallas import tpu_sc as plsc

T = 2048
D = 2048
E = 8
W = 1024
K = 2
BLK = 128
S = 5120
NB = S // BLK

NC = 2
NS = 16
NW = NC * NS

_HD = D // 2


def _bf16_bits(xf32):
    t = lax.bitcast_convert_type(xf32, jnp.int32)
    carry = jnp.bitwise_and(lax.shift_right_logical(t, 16), 1)
    return lax.shift_right_logical(t + 32767 + carry, 16)


def _router_body(x_ref, wr_ref, p0_ref, p1_ref, cw0_ref, cw1_ref,
                 blke_ref, z_ref, xp_ref):
    x = x_ref[...]
    wr = wr_ref[...]
    lo = _bf16_bits(x[:, :_HD])
    hi = _bf16_bits(x[:, _HD:])
    xp_ref[...] = jnp.bitwise_or(lo, lax.shift_left(hi, 16))
    logits = lax.dot_general(x, wr, (((1,), (0,)), ((), ())),
                             preferred_element_type=jnp.float32)
    eidx = lax.broadcasted_iota(jnp.int32, (T, E), 1)
    m1 = jnp.max(logits, axis=1, keepdims=True)
    a1 = jnp.min(jnp.where(logits == m1, eidx, E), axis=1, keepdims=True)
    logits2 = jnp.where(eidx == a1, -jnp.inf, logits)
    m2 = jnp.max(logits2, axis=1, keepdims=True)
    a2 = jnp.min(jnp.where(logits2 == m2, eidx, E), axis=1, keepdims=True)
    s1 = jax.nn.sigmoid(m1)
    s2 = jax.nn.sigmoid(m2)
    denom = s1 + s2 + 1e-20
    cw0_ref[...] = s1 / denom
    cw1_ref[...] = s2 / denom
    oh1 = eidx == a1
    oh2 = eidx == a2
    ohsum = oh1.astype(jnp.float32) + oh2.astype(jnp.float32)
    cnt = jnp.sum(ohsum, axis=0, keepdims=True)
    pc = jnp.ceil(cnt / BLK) * BLK
    ei = lax.broadcasted_iota(jnp.int32, (E, E), 0)
    ej = lax.broadcasted_iota(jnp.int32, (E, E), 1)
    strict = (ei < ej).astype(jnp.float32)
    starts = lax.dot_general(pc, strict, (((1,), (0,)), ((), ())),
                             preferred_element_type=jnp.float32)
    ti = lax.broadcasted_iota(jnp.int32, (T, 1), 0)
    tj = lax.broadcasted_iota(jnp.int32, (1, T), 1)
    lmask = (tj < ti).astype(jnp.bfloat16)
    cex = lax.dot_general(lmask, ohsum.astype(jnp.bfloat16),
                          (((1,), (0,)), ((), ())),
                          preferred_element_type=jnp.float32)
    pos_base = cex + starts
    p0_ref[...] = jnp.sum(jnp.where(oh1, pos_base, 0.0), axis=1,
                          keepdims=True).astype(jnp.int32)
    p1_ref[...] = jnp.sum(jnp.where(oh2, pos_base, 0.0), axis=1,
                          keepdims=True).astype(jnp.int32)
    bi = lax.broadcasted_iota(jnp.int32, (NB, E), 0).astype(jnp.float32) * BLK
    blke_ref[...] = (jnp.sum((bi >= starts).astype(jnp.int32), axis=1,
                             keepdims=True) - 1)
    mx = jnp.max(logits, axis=1, keepdims=True)
    lse = mx + jnp.log(jnp.sum(jnp.exp(logits - mx), axis=1, keepdims=True))
    z_ref[...] = jnp.mean(lse * lse).reshape(1, 1)


def _router_call(x_flat, W_router):
    return pl.pallas_call(
        _router_body,
        out_shape=(
            jax.ShapeDtypeStruct((T, 1), jnp.int32),
            jax.ShapeDtypeStruct((T, 1), jnp.int32),
            jax.ShapeDtypeStruct((T, 1), jnp.float32),
            jax.ShapeDtypeStruct((T, 1), jnp.float32),
            jax.ShapeDtypeStruct((NB, 1), jnp.int32),
            jax.ShapeDtypeStruct((1, 1), jnp.float32),
            jax.ShapeDtypeStruct((T, _HD), jnp.int32),
        ),
    )(x_flat, W_router)


def _slots_body(p0_ref, p1_ref, cw0_ref, cw1_ref, tok_ref, cw_ref):
    j = pl.program_id(0)
    p0r = p0_ref[...]
    p1r = p1_ref[...]
    cw0r = cw0_ref[...]
    cw1r = cw1_ref[...]
    sid = lax.broadcasted_iota(jnp.int32, (BLK, T), 0) + j * BLK
    tok = lax.broadcasted_iota(jnp.int32, (BLK, T), 1)
    m0 = sid == p0r
    m1 = sid == p1r
    tok_ref[...] = (jnp.sum(jnp.where(m0, tok, 0), axis=1, keepdims=True)
                    + jnp.sum(jnp.where(m1, tok, 0), axis=1, keepdims=True))
    cw_ref[...] = (jnp.sum(jnp.where(m0, cw0r, 0.0), axis=1, keepdims=True)
                   + jnp.sum(jnp.where(m1, cw1r, 0.0), axis=1, keepdims=True))


def _slots_call(p0_row, p1_row, cw0_row, cw1_row):
    return pl.pallas_call(
        _slots_body,
        grid=(NB,),
        in_specs=[pl.BlockSpec((1, T), lambda j: (0, 0))] * 4,
        out_specs=(
            pl.BlockSpec((BLK, 1), lambda j: (j, 0)),
            pl.BlockSpec((BLK, 1), lambda j: (j, 0)),
        ),
        out_shape=(
            jax.ShapeDtypeStruct((S, 1), jnp.int32),
            jax.ShapeDtypeStruct((S, 1), jnp.float32),
        ),
    )(p0_row, p1_row, cw0_row, cw1_row)


SH = S // 2
_CROWS = 16
_RPW = SH // NW
_NCH = _RPW // _CROWS
_NBUF = 4


def _gather_body(lo, x_hbm, idx_hbm, out_hbm, idx_v, b0, b1, b2, b3,
                 sg0, sg1, sg2, sg3, sw0, sw1, sw2, sw3):
    wid = lax.axis_index("s") * NC + lax.axis_index("c")
    base = wid * _RPW
    pltpu.sync_copy(idx_hbm.at[pl.ds(lo + base, _RPW)], idx_v)
    bufs = [b0, b1, b2, b3]
    gsems = [sg0, sg1, sg2, sg3]
    wsems = [sw0, sw1, sw2, sw3]

    def gath(c):
        return pltpu.async_copy(
            x_hbm.at[idx_v.at[pl.ds(c * _CROWS, _CROWS)]],
            bufs[c % _NBUF], gsems[c % _NBUF])

    def wr(c):
        return pltpu.async_copy(
            bufs[c % _NBUF], out_hbm.at[pl.ds(base + c * _CROWS, _CROWS)],
            wsems[c % _NBUF])

    gd = [None] * _NCH
    wd = [None] * _NCH
    for c in range(_NBUF):
        gd[c] = gath(c)
    for c in range(_NCH):
        gd[c].wait()
        wd[c] = wr(c)
        n = c + _NBUF
        if n < _NCH:
            wd[c].wait()
            gd[n] = gath(n)
    for c in range(_NCH - _NBUF, _NCH):
        wd[c].wait()


def _gather_rows(x_packed, slot_tok, lo):
    mesh = plsc.VectorSubcoreMesh(core_axis_name="c", subcore_axis_name="s",
                                  num_cores=NC, num_subcores=NS)
    fn = pl.kernel(
        functools.partial(_gather_body, lo),
        out_type=jax.ShapeDtypeStruct((SH, _HD), jnp.int32),
        mesh=mesh,
        scratch_types=(
            [pltpu.VMEM((_RPW,), jnp.int32)]
            + [pltpu.VMEM((_CROWS, _HD), jnp.int32)] * _NBUF
            + [pltpu.SemaphoreType.DMA] * (2 * _NBUF)
        ),
    )
    return fn(x_packed, slot_tok)


def _mlp_body(blke_ref, xs_ref, w1_ref, w2_ref, cw_ref, out_ref):
    xp = xs_ref[...]
    xlo = lax.bitcast_convert_type(lax.shift_left(xp, 16), jnp.float32)
    xhi = lax.bitcast_convert_type(
        jnp.bitwise_and(xp, jnp.int32(-65536)), jnp.float32)
    xs = jnp.concatenate([xlo, xhi], axis=1)
    h = lax.dot_general(xs, w1_ref[...], (((1,), (0,)), ((), ())),
                        preferred_element_type=jnp.float32)
    h = jnp.square(jnp.maximum(h, 0.0))
    o = lax.dot_general(h, w2_ref[...], (((1,), (0,)), ((), ())),
                        preferred_element_type=jnp.float32)
    out_ref[...] = o * cw_ref[...]


def _mlp_half1(blke, xs_packed, w1, w2, slot_cw):
    grid_spec = pltpu.PrefetchScalarGridSpec(
        num_scalar_prefetch=1,
        grid=(NB // 2,),
        in_specs=[
            pl.BlockSpec((BLK, _HD), lambda b, blke: (b, 0)),
            pl.BlockSpec((D, W), lambda b, blke: (0, blke[b])),
            pl.BlockSpec((W, D), lambda b, blke: (blke[b], 0)),
            pl.BlockSpec((BLK, 1), lambda b, blke: (b, 0)),
        ],
        out_specs=pl.BlockSpec((BLK, D), lambda b, blke: (b, 0)),
    )
    return pl.pallas_call(
        _mlp_body,
        grid_spec=grid_spec,
        out_shape=jax.ShapeDtypeStruct((S, D), jnp.float32),
    )(blke, xs_packed, w1, w2, slot_cw)


def _mlp_body2(blke_ref, xs_ref, w1_ref, w2_ref, cw_ref, prev_ref, out_ref):
    _mlp_body(blke_ref, xs_ref, w1_ref, w2_ref, cw_ref, out_ref)


def _mlp_half2(blke, xs_packed, w1, w2, slot_cw, partial_prev):
    nb0 = NB // 2
    grid_spec = pltpu.PrefetchScalarGridSpec(
        num_scalar_prefetch=1,
        grid=(NB - nb0,),
        in_specs=[
            pl.BlockSpec((BLK, _HD), lambda b, blke: (b, 0)),
            pl.BlockSpec((D, W), lambda b, blke: (0, blke[nb0 + b])),
            pl.BlockSpec((W, D), lambda b, blke: (blke[nb0 + b], 0)),
            pl.BlockSpec((BLK, 1), lambda b, blke: (nb0 + b, 0)),
            pl.BlockSpec((8, 128), lambda b, blke: (0, 0)),
        ],
        out_specs=pl.BlockSpec((BLK, D), lambda b, blke: (nb0 + b, 0)),
    )
    return pl.pallas_call(
        _mlp_body2,
        grid_spec=grid_spec,
        out_shape=jax.ShapeDtypeStruct((S, D), jnp.float32),
        input_output_aliases={5: 0},
    )(blke, xs_packed, w1, w2, slot_cw, partial_prev)


_CTOK = 8
_TPW = T // NW
_NCHC = _TPW // _CTOK
_NBUFC = 2


def _combine_body(part_hbm, p0_hbm, p1_hbm, out_hbm, p0_v, p1_v,
                  a0, a1, g0, g1, sa0, sa1, sg0, sg1, sw0, sw1):
    wid = lax.axis_index("s") * NC + lax.axis_index("c")
    base = wid * _TPW
    pltpu.sync_copy(p0_hbm.at[pl.ds(base, _TPW)], p0_v)
    pltpu.sync_copy(p1_hbm.at[pl.ds(base, _TPW)], p1_v)
    accs, gs = [a0, a1], [g0, g1]
    asems, gsems, wsems = [sa0, sa1], [sg0, sg1], [sw0, sw1]

    def gath(c):
        k = c % _NBUFC
        sl = pl.ds(c * _CTOK, _CTOK)
        return (pltpu.async_copy(part_hbm.at[p0_v.at[sl]], accs[k], asems[k]),
                pltpu.async_copy(part_hbm.at[p1_v.at[sl]], gs[k], gsems[k]))

    def wr(c):
        k = c % _NBUFC
        return pltpu.async_copy(
            accs[k], out_hbm.at[pl.ds(base + c * _CTOK, _CTOK)], wsems[k])

    gd = [None] * _NCHC
    wd = [None] * _NCHC
    for c in range(_NBUFC):
        gd[c] = gath(c)
    for c in range(_NCHC):
        k = c % _NBUFC
        gd[c][0].wait()
        gd[c][1].wait()
        acc_v, g_v = accs[k], gs[k]
        for r in range(_CTOK):
            def add16(ci, carry2, r=r, acc_v=acc_v, g_v=g_v):
                sl = pl.ds(ci * 16, 16)
                acc_v[r, sl] = acc_v[r, sl] + g_v[r, sl]
                return carry2
            lax.fori_loop(0, D // 16, add16, 0, unroll=8)
        wd[c] = wr(c)
        n = c + _NBUFC
        if n < _NCHC:
            wd[c].wait()
            gd[n] = gath(n)
    for c in range(_NCHC - _NBUFC, _NCHC):
        wd[c].wait()


def _combine_rows(partial, p0, p1):
    mesh = plsc.VectorSubcoreMesh(core_axis_name="c", subcore_axis_name="s",
                                  num_cores=NC, num_subcores=NS)
    fn = pl.kernel(
        _combine_body,
        out_type=jax.ShapeDtypeStruct((T, D), jnp.float32),
        mesh=mesh,
        scratch_types=(
            [pltpu.VMEM((_TPW,), jnp.int32)] * 2
            + [pltpu.VMEM((_CTOK, D), jnp.float32)] * (2 * _NBUFC)
            + [pltpu.SemaphoreType.DMA] * (3 * _NBUFC)
        ),
    )
    return fn(partial, p0, p1)


def kernel(x, W_router, w1, w2):
    b, s, d = x.shape
    x_flat = x.reshape(T, D)
    pos0, pos1, cw0, cw1, blke, z, x_packed = _router_call(x_flat, W_router)
    slot_tok_col, slot_cw = _slots_call(pos0.reshape(1, T), pos1.reshape(1, T),
                                        cw0.reshape(1, T), cw1.reshape(1, T))
    slot_tok = slot_tok_col.reshape(S)
    blke_v = blke.reshape(NB)
    xs1 = _gather_rows(x_packed, slot_tok, 0)
    xs2 = _gather_rows(x_packed, slot_tok, SH)
    partial1 = _mlp_half1(blke_v, xs1, w1, w2, slot_cw)
    partial = _mlp_half2(blke_v, xs2, w1, w2, slot_cw, partial1)
    out_flat = _combine_rows(partial, pos0.reshape(T), pos1.reshape(T))
    return out_flat.reshape(b, s, d), z.reshape(())
